# Initial kernel scaffold; baseline (speedup 1.0000x reference)
import jax
import jax.numpy as jnp
from jax.experimental import pallas as pl


def kernel(x, edge_index, edge_attr, W1, b1, W2, b2, Wc, bc):
    raise NotImplementedError("write your pallas kernel here")


# speedup vs baseline: 5.5584x; 5.5584x over previous
import functools

import jax
import jax.numpy as jnp
from jax import lax
from jax.experimental import pallas as pl
from jax.experimental.pallas import tpu as pltpu
from jax.experimental.pallas import tpu_sc as plsc

N = 10000
NPAD = 10240
E = 160000
EPAD = 163840
DIN = 256
DH = 512
CW = 128
NCHUNK = DH // CW
NC, NS = 2, 16
TPC = 4

_f32 = jnp.float32


def _sc_mesh():
    return plsc.VectorSubcoreMesh(core_axis_name="c", subcore_axis_name="s")


_SC_PARAMS = pltpu.CompilerParams(needs_layout_passes=False)


def _deg_body(dst_hbm, out_hbm, idx_v, ones_v, zb_v, acc_sh):
    c = lax.axis_index("c")
    s = lax.axis_index("s")
    wid = c * NS + s
    for i in range(8):
        ones_v[pl.ds(i * 16, 16)] = jnp.full((16,), 1.0, _f32)
    for i in range(40):
        zb_v[pl.ds(i * 16, 16)] = jnp.zeros((16,), _f32)
    pltpu.sync_copy(dst_hbm.at[wid], idx_v)
    pltpu.sync_copy(zb_v, acc_sh.at[pl.ds(s * 640, 640)])
    plsc.subcore_barrier()

    @pl.loop(0, 40)
    def _(j):
        pltpu.sync_copy(ones_v, acc_sh.at[idx_v.at[j]], add=True)

    plsc.subcore_barrier()
    pltpu.sync_copy(acc_sh.at[pl.ds(s * 640, 640)],
                    out_hbm.at[c, pl.ds(s * 640, 640)])


def _deg_call(dst32):
    k = pl.kernel(
        _deg_body,
        out_type=jax.ShapeDtypeStruct((NC, NPAD), _f32),
        mesh=_sc_mesh(),
        compiler_params=_SC_PARAMS,
        scratch_types=[
            pltpu.VMEM((40, 128), jnp.int32),
            pltpu.VMEM((128,), _f32),
            pltpu.VMEM((640,), _f32),
            pltpu.VMEM_SHARED((NPAD,), _f32),
        ],
    )
    return k(dst32)


def _agg_body(y0, y1, y2, y3, srcg, dstg, a0, a1, a2, a3,
              sidx, didx, gbuf, zb, sem, acc_sh):
    c = lax.axis_index("c")
    s = lax.axis_index("s")
    pltpu.sync_copy(srcg.at[s], sidx)
    pltpu.sync_copy(dstg.at[s], didx)

    @pl.loop(0, 64)
    def _(r):
        for kk in range(8):
            zb[r, pl.ds(kk * 16, 16)] = jnp.zeros((16,), _f32)

    def run_chunk(ytab, aout):
        @pl.loop(0, 10)
        def _(i):
            pltpu.sync_copy(zb, acc_sh.at[pl.ds(s * 640 + i * 64, 64)])

        plsc.subcore_barrier()

        @pl.loop(0, 80)
        def _(j):
            pltpu.async_copy(ytab.at[sidx.at[j]], gbuf, sem).wait()
            pltpu.sync_copy(gbuf, acc_sh.at[didx.at[j]], add=True)

        plsc.subcore_barrier()
        pltpu.sync_copy(acc_sh.at[pl.ds(s * 640, 640)],
                        aout.at[pl.ds(s * 640, 640)])
        plsc.subcore_barrier()

    @pl.when(c == 0)
    def _():
        run_chunk(y0, a0)
        run_chunk(y1, a1)

    @pl.when(c == 1)
    def _():
        run_chunk(y2, a2)
        run_chunk(y3, a3)


def _agg_call(ycs, src16, dst16):
    k = pl.kernel(
        _agg_body,
        out_type=[jax.ShapeDtypeStruct((NPAD, CW), _f32)] * 4,
        mesh=_sc_mesh(),
        compiler_params=_SC_PARAMS,
        scratch_types=[
            pltpu.VMEM((80, 128), jnp.int32),
            pltpu.VMEM((80, 128), jnp.int32),
            pltpu.VMEM((128, CW), _f32),
            pltpu.VMEM((64, CW), _f32),
            pltpu.SemaphoreType.DMA,
            pltpu.VMEM_SHARED((NPAD, CW), _f32),
        ],
    )
    return k(ycs[0], ycs[1], ycs[2], ycs[3], src16, dst16)


def _cls_body(atab, btab, rowg, colg, oute, atv, btv, ridx, cidx, obuf):
    c = lax.axis_index("c")
    s = lax.axis_index("s")
    wid = c * NS + s
    pltpu.sync_copy(rowg.at[wid], ridx)
    pltpu.sync_copy(colg.at[wid], cidx)
    pltpu.sync_copy(atab, atv)
    pltpu.sync_copy(btab, btv)
    lanes = lax.iota(jnp.int32, 16)

    @pl.loop(0, 40)
    def _(j):
        for k in range(8):
            eA = ridx[pl.ds(j * 128 + k * 16, 16)] * TPC
            eB = cidx[pl.ds(j * 128 + k * 16, 16)] * TPC
            for col in range(TPC):
                sv = (plsc.load_gather(atv, [eA + col]) +
                      plsc.load_gather(btv, [eB + col]))
                plsc.store_scatter(obuf, [lanes * TPC + k * 16 * TPC + col], sv)
        pltpu.sync_copy(obuf, oute.at[pl.ds((wid * 5120 + j * 128) * TPC,
                                            128 * TPC)])


def _cls_call(atab, btab, row32, col32):
    k = pl.kernel(
        _cls_body,
        out_type=jax.ShapeDtypeStruct((EPAD * TPC,), _f32),
        mesh=_sc_mesh(),
        compiler_params=_SC_PARAMS,
        scratch_types=[
            pltpu.VMEM((N * TPC,), _f32),
            pltpu.VMEM((N * TPC,), _f32),
            pltpu.VMEM((5120,), jnp.int32),
            pltpu.VMEM((5120,), jnp.int32),
            pltpu.VMEM((128 * TPC,), _f32),
        ],
    )
    return k(atab, btab, row32, col32)


def _dinv_body(degp_ref, out_ref):
    d = 1.0 + degp_ref[0, :] + degp_ref[1, :]
    out_ref[...] = lax.rsqrt(d)[:, None]


def _dinv_call(degp):
    return pl.pallas_call(
        _dinv_body,
        grid=(10,),
        in_specs=[pl.BlockSpec((NC, 1024), lambda i: (0, i))],
        out_specs=pl.BlockSpec((1024, 1), lambda i: (i, 0)),
        out_shape=jax.ShapeDtypeStruct((NPAD, 1), _f32),
    )(degp)


def _mm1_body(x_ref, w_ref, dv_ref, o0, o1, o2, o3):
    y = jnp.dot(x_ref[...], w_ref[...], preferred_element_type=_f32)
    y = y * dv_ref[...]
    o0[...] = y[:, 0:128]
    o1[...] = y[:, 128:256]
    o2[...] = y[:, 256:384]
    o3[...] = y[:, 384:512]


def _mm1_call(x, W1, dinv):
    return pl.pallas_call(
        _mm1_body,
        grid=(25,),
        in_specs=[
            pl.BlockSpec((400, DIN), lambda i: (i, 0)),
            pl.BlockSpec((DIN, DH), lambda i: (0, 0)),
            pl.BlockSpec((400, 1), lambda i: (i, 0)),
        ],
        out_specs=[pl.BlockSpec((400, CW), lambda i: (i, 0))] * 4,
        out_shape=[jax.ShapeDtypeStruct((N, CW), _f32)] * 4,
    )(x, W1, dinv)


def _mm2_body(dv_ref, b_ref, w_ref, a0, a1, a2, a3, y0, y1, y2, y3,
              o0, o1, o2, o3):
    dv = dv_ref[...]
    h = jnp.concatenate(
        [a0[...] + y0[...], a1[...] + y1[...],
         a2[...] + y2[...], a3[...] + y3[...]], axis=1)
    h = jnp.tanh(h * dv + b_ref[...])
    y = jnp.dot(h, w_ref[...], preferred_element_type=_f32) * dv
    o0[...] = y[:, 0:128]
    o1[...] = y[:, 128:256]
    o2[...] = y[:, 256:384]
    o3[...] = y[:, 384:512]


def _mm2_call(dinv, b1r, W2, aggs, ycs):
    return pl.pallas_call(
        _mm2_body,
        grid=(25,),
        in_specs=[
            pl.BlockSpec((400, 1), lambda i: (i, 0)),
            pl.BlockSpec((1, DH), lambda i: (0, 0)),
            pl.BlockSpec((DH, DH), lambda i: (0, 0)),
        ] + [pl.BlockSpec((400, CW), lambda i: (i, 0))] * 8,
        out_specs=[pl.BlockSpec((400, CW), lambda i: (i, 0))] * 4,
        out_shape=[jax.ShapeDtypeStruct((N, CW), _f32)] * 4,
    )(dinv, b1r, W2, *aggs, *ycs)


def _mm3_body(dv_ref, b_ref, wa_ref, wb_ref, bca_ref,
              a0, a1, a2, a3, y0, y1, y2, y3, oa, ob):
    dv = dv_ref[...]
    h = jnp.concatenate(
        [a0[...] + y0[...], a1[...] + y1[...],
         a2[...] + y2[...], a3[...] + y3[...]], axis=1)
    h = jnp.clip(h * dv + b_ref[...], 0.0, 6.0)
    oa[...] = jnp.dot(h, wa_ref[...], preferred_element_type=_f32) + bca_ref[...]
    ob[...] = jnp.dot(h, wb_ref[...], preferred_element_type=_f32)


def _mm3_call(dinv, b2r, wa, wb, bca, aggs, ycs):
    return pl.pallas_call(
        _mm3_body,
        grid=(25,),
        in_specs=[
            pl.BlockSpec((400, 1), lambda i: (i, 0)),
            pl.BlockSpec((1, DH), lambda i: (0, 0)),
            pl.BlockSpec((DH, TPC), lambda i: (0, 0)),
            pl.BlockSpec((DH, TPC), lambda i: (0, 0)),
            pl.BlockSpec((1, TPC), lambda i: (0, 0)),
        ] + [pl.BlockSpec((400, CW), lambda i: (i, 0))] * 8,
        out_specs=[pl.BlockSpec((400, TPC), lambda i: (i, 0))] * 2,
        out_shape=[jax.ShapeDtypeStruct((N, TPC), _f32)] * 2,
    )(dinv, b2r, wa, wb, bca, *aggs, *ycs)


def kernel(x, edge_index, edge_attr, W1, b1, W2, b2, Wc, bc):
    row = edge_index[0]
    col = edge_index[1]
    pad = EPAD - E
    srcp = jnp.concatenate([row, jnp.zeros((pad,), jnp.int32)])
    dstp = jnp.concatenate([col, jnp.full((pad,), N, jnp.int32)])
    colg = jnp.concatenate([col, jnp.zeros((pad,), jnp.int32)])
    dst32 = dstp.reshape(32, 40, 128)
    src16 = srcp.reshape(16, 80, 128)
    dst16 = dstp.reshape(16, 80, 128)
    src32 = srcp.reshape(32, 5120)
    col32 = colg.reshape(32, 5120)

    b1r = b1.reshape(1, DH)
    b2r = b2.reshape(1, DH)
    wa = jnp.pad(Wc[:DH], ((0, 0), (0, TPC - Wc.shape[1])))
    wb = jnp.pad(Wc[DH:], ((0, 0), (0, TPC - Wc.shape[1])))
    bca = jnp.pad(bc, (0, TPC - bc.shape[0])).reshape(1, TPC)

    degp = _deg_call(dst32)
    dinv = _dinv_call(degp)
    y1c = _mm1_call(x, W1, dinv)
    agg1 = _agg_call(y1c, src16, dst16)
    y2c = _mm2_call(dinv, b1r, W2, agg1, y1c)
    agg2 = _agg_call(y2c, src16, dst16)
    atab, btab = _mm3_call(dinv, b2r, wa, wb, bca, agg2, y2c)
    oute = _cls_call(atab.reshape(N * TPC), btab.reshape(N * TPC), src32, col32)
    return oute.reshape(EPAD, TPC)[:E, :3]


# speedup vs baseline: 12.7364x; 2.2914x over previous
import functools

import jax
import jax.numpy as jnp
from jax import lax
from jax.experimental import pallas as pl
from jax.experimental.pallas import tpu as pltpu
from jax.experimental.pallas import tpu_sc as plsc

N = 10000
NPAD = 10240
E = 160000
EPAD = 163840
DIN = 256
DH = 512
CW = 128
NCHUNK = DH // CW
NC, NS = 2, 16
TPC = 4

_f32 = jnp.float32


def _sc_mesh():
    return plsc.VectorSubcoreMesh(core_axis_name="c", subcore_axis_name="s")


_SC_PARAMS = pltpu.CompilerParams(needs_layout_passes=False)


def _deg_body(dst_hbm, out_hbm, idx_v, ones_v, zb_v, acc_sh):
    c = lax.axis_index("c")
    s = lax.axis_index("s")
    wid = c * NS + s
    for i in range(8):
        ones_v[pl.ds(i * 16, 16)] = jnp.full((16,), 1.0, _f32)
    for i in range(40):
        zb_v[pl.ds(i * 16, 16)] = jnp.zeros((16,), _f32)
    pltpu.sync_copy(dst_hbm.at[wid], idx_v)
    pltpu.sync_copy(zb_v, acc_sh.at[pl.ds(s * 640, 640)])
    plsc.subcore_barrier()

    @pl.loop(0, 40)
    def _(j):
        pltpu.sync_copy(ones_v, acc_sh.at[idx_v.at[j]], add=True)

    plsc.subcore_barrier()
    pltpu.sync_copy(acc_sh.at[pl.ds(s * 640, 640)],
                    out_hbm.at[c, pl.ds(s * 640, 640)])


def _deg_call(dst32):
    k = pl.kernel(
        _deg_body,
        out_type=jax.ShapeDtypeStruct((NC, NPAD), _f32),
        mesh=_sc_mesh(),
        compiler_params=_SC_PARAMS,
        scratch_types=[
            pltpu.VMEM((40, 128), jnp.int32),
            pltpu.VMEM((128,), _f32),
            pltpu.VMEM((640,), _f32),
            pltpu.VMEM_SHARED((NPAD,), _f32),
        ],
    )
    return k(dst32)


_NT = 80


def _agg_body(y0, y1, y2, y3, idxsd, a0, a1, a2, a3,
              islots, gb0, gb1, zb, isems, gsems, ssems, acc_sh):
    gbufs = (gb0, gb1)
    c = lax.axis_index("c")
    s = lax.axis_index("s")

    @pl.loop(0, 32)
    def _(r):
        for kk in range(8):
            zb[r, pl.ds(kk * 16, 16)] = jnp.zeros((16,), _f32)

    def run_chunk(ytab, aout):
        @pl.loop(0, 20)
        def _(i):
            pltpu.sync_copy(zb, acc_sh.at[pl.ds(s * 640 + i * 32, 32)])

        plsc.subcore_barrier()

        def start_idx(t, k):
            pltpu.async_copy(idxsd.at[s, t], islots.at[k], isems[k])

        def wait_idx(t, k):
            pltpu.make_async_copy(idxsd.at[s, t], islots.at[k],
                                  isems[k]).wait()

        def start_gather(t, b, k):
            pltpu.async_copy(ytab.at[islots.at[k, 0]], gbufs[b], gsems[b])

        def wait_gather(b, k):
            pltpu.make_async_copy(ytab.at[islots.at[k, 0]], gbufs[b],
                                  gsems[b]).wait()

        def start_scatter(b, k):
            pltpu.async_copy(gbufs[b], acc_sh.at[islots.at[k, 1]], ssems[b],
                             add=True)

        def wait_scatter(b, k):
            pltpu.make_async_copy(gbufs[b], acc_sh.at[islots.at[k, 1]],
                                  ssems[b]).wait()

        for k in range(4):
            start_idx(k, k)
        wait_idx(0, 0)
        start_gather(0, 0, 0)

        @pl.loop(0, _NT // 4)
        def _(g):
            for u in range(4):
                b = u % 2
                k0 = u
                t = g * 4 + u
                if u == 0:
                    @pl.when(g > 0)
                    def _():
                        wait_scatter(1 - b, (u - 1) % 4)
                else:
                    wait_scatter(1 - b, (u - 1) % 4)
                if u == 3:
                    @pl.when(g < _NT // 4 - 1)
                    def _():
                        wait_idx(t + 1, (u + 1) % 4)
                        start_gather(t + 1, 1 - b, (u + 1) % 4)
                else:
                    wait_idx(t + 1, (u + 1) % 4)
                    start_gather(t + 1, 1 - b, (u + 1) % 4)
                wait_gather(b, k0)
                start_scatter(b, k0)
                if u == 0:
                    @pl.when(g > 0)
                    def _():
                        start_idx(t + 3, (u + 3) % 4)
                else:
                    @pl.when(t + 3 < _NT)
                    def _():
                        start_idx(t + 3, (u + 3) % 4)

        wait_scatter(1, 3)
        plsc.subcore_barrier()
        pltpu.sync_copy(acc_sh.at[pl.ds(s * 640, 640)],
                        aout.at[pl.ds(s * 640, 640)])
        plsc.subcore_barrier()

    @pl.when(c == 0)
    def _():
        run_chunk(y0, a0)
        run_chunk(y1, a1)

    @pl.when(c == 1)
    def _():
        run_chunk(y2, a2)
        run_chunk(y3, a3)


def _agg_call(ycs, idxsd):
    k = pl.kernel(
        _agg_body,
        out_type=[jax.ShapeDtypeStruct((NPAD, CW), _f32)] * 4,
        mesh=_sc_mesh(),
        compiler_params=_SC_PARAMS,
        scratch_types=[
            pltpu.VMEM((4, 2, 128), jnp.int32),
            pltpu.VMEM((128, CW), _f32),
            pltpu.VMEM((128, CW), _f32),
            pltpu.VMEM((32, CW), _f32),
            [pltpu.SemaphoreType.DMA] * 4,
            [pltpu.SemaphoreType.DMA] * 2,
            [pltpu.SemaphoreType.DMA] * 2,
            pltpu.VMEM_SHARED((NPAD, CW), _f32),
        ],
    )
    return k(ycs[0], ycs[1], ycs[2], ycs[3], idxsd)


def _cls_body(atab, btab, rowg, colg, oute, atv, btv, ridx, cidx, obuf):
    c = lax.axis_index("c")
    s = lax.axis_index("s")
    wid = c * NS + s
    pltpu.sync_copy(rowg.at[wid], ridx)
    pltpu.sync_copy(colg.at[wid], cidx)
    pltpu.sync_copy(atab, atv)
    pltpu.sync_copy(btab, btv)
    lanes = lax.iota(jnp.int32, 16)

    @pl.loop(0, 40)
    def _(j):
        for k in range(8):
            eA = ridx[pl.ds(j * 128 + k * 16, 16)] * TPC
            eB = cidx[pl.ds(j * 128 + k * 16, 16)] * TPC
            for col in range(TPC):
                sv = (plsc.load_gather(atv, [eA + col]) +
                      plsc.load_gather(btv, [eB + col]))
                plsc.store_scatter(obuf, [lanes * TPC + k * 16 * TPC + col], sv)
        pltpu.sync_copy(obuf, oute.at[pl.ds((wid * 5120 + j * 128) * TPC,
                                            128 * TPC)])


def _cls_call(atab, btab, row32, col32):
    k = pl.kernel(
        _cls_body,
        out_type=jax.ShapeDtypeStruct((EPAD * TPC,), _f32),
        mesh=_sc_mesh(),
        compiler_params=_SC_PARAMS,
        scratch_types=[
            pltpu.VMEM((N * TPC,), _f32),
            pltpu.VMEM((N * TPC,), _f32),
            pltpu.VMEM((5120,), jnp.int32),
            pltpu.VMEM((5120,), jnp.int32),
            pltpu.VMEM((128 * TPC,), _f32),
        ],
    )
    return k(atab, btab, row32, col32)


def _dinv_body(degp_ref, out_ref):
    d = 1.0 + degp_ref[0, :] + degp_ref[1, :]
    out_ref[...] = lax.rsqrt(d)[:, None]


def _dinv_call(degp):
    return pl.pallas_call(
        _dinv_body,
        grid=(10,),
        in_specs=[pl.BlockSpec((NC, 1024), lambda i: (0, i))],
        out_specs=pl.BlockSpec((1024, 1), lambda i: (i, 0)),
        out_shape=jax.ShapeDtypeStruct((NPAD, 1), _f32),
    )(degp)


def _mm1_body(x_ref, w_ref, dv_ref, o0, o1, o2, o3):
    y = jnp.dot(x_ref[...], w_ref[...], preferred_element_type=_f32)
    y = y * dv_ref[...]
    o0[...] = y[:, 0:128]
    o1[...] = y[:, 128:256]
    o2[...] = y[:, 256:384]
    o3[...] = y[:, 384:512]


def _mm1_call(x, W1, dinv):
    return pl.pallas_call(
        _mm1_body,
        grid=(25,),
        in_specs=[
            pl.BlockSpec((400, DIN), lambda i: (i, 0)),
            pl.BlockSpec((DIN, DH), lambda i: (0, 0)),
            pl.BlockSpec((400, 1), lambda i: (i, 0)),
        ],
        out_specs=[pl.BlockSpec((400, CW), lambda i: (i, 0))] * 4,
        out_shape=[jax.ShapeDtypeStruct((N, CW), _f32)] * 4,
    )(x, W1, dinv)


def _mm2_body(dv_ref, b_ref, w_ref, a0, a1, a2, a3, y0, y1, y2, y3,
              o0, o1, o2, o3):
    dv = dv_ref[...]
    h = jnp.concatenate(
        [a0[...] + y0[...], a1[...] + y1[...],
         a2[...] + y2[...], a3[...] + y3[...]], axis=1)
    h = jnp.tanh(h * dv + b_ref[...])
    y = jnp.dot(h, w_ref[...], preferred_element_type=_f32) * dv
    o0[...] = y[:, 0:128]
    o1[...] = y[:, 128:256]
    o2[...] = y[:, 256:384]
    o3[...] = y[:, 384:512]


def _mm2_call(dinv, b1r, W2, aggs, ycs):
    return pl.pallas_call(
        _mm2_body,
        grid=(25,),
        in_specs=[
            pl.BlockSpec((400, 1), lambda i: (i, 0)),
            pl.BlockSpec((1, DH), lambda i: (0, 0)),
            pl.BlockSpec((DH, DH), lambda i: (0, 0)),
        ] + [pl.BlockSpec((400, CW), lambda i: (i, 0))] * 8,
        out_specs=[pl.BlockSpec((400, CW), lambda i: (i, 0))] * 4,
        out_shape=[jax.ShapeDtypeStruct((N, CW), _f32)] * 4,
    )(dinv, b1r, W2, *aggs, *ycs)


def _mm3_body(dv_ref, b_ref, wa_ref, wb_ref, bca_ref,
              a0, a1, a2, a3, y0, y1, y2, y3, oa, ob):
    dv = dv_ref[...]
    h = jnp.concatenate(
        [a0[...] + y0[...], a1[...] + y1[...],
         a2[...] + y2[...], a3[...] + y3[...]], axis=1)
    h = jnp.clip(h * dv + b_ref[...], 0.0, 6.0)
    oa[...] = jnp.dot(h, wa_ref[...], preferred_element_type=_f32) + bca_ref[...]
    ob[...] = jnp.dot(h, wb_ref[...], preferred_element_type=_f32)


def _mm3_call(dinv, b2r, wa, wb, bca, aggs, ycs):
    return pl.pallas_call(
        _mm3_body,
        grid=(25,),
        in_specs=[
            pl.BlockSpec((400, 1), lambda i: (i, 0)),
            pl.BlockSpec((1, DH), lambda i: (0, 0)),
            pl.BlockSpec((DH, TPC), lambda i: (0, 0)),
            pl.BlockSpec((DH, TPC), lambda i: (0, 0)),
            pl.BlockSpec((1, TPC), lambda i: (0, 0)),
        ] + [pl.BlockSpec((400, CW), lambda i: (i, 0))] * 8,
        out_specs=[pl.BlockSpec((400, TPC), lambda i: (i, 0))] * 2,
        out_shape=[jax.ShapeDtypeStruct((N, TPC), _f32)] * 2,
    )(dinv, b2r, wa, wb, bca, *aggs, *ycs)


def kernel(x, edge_index, edge_attr, W1, b1, W2, b2, Wc, bc):
    row = edge_index[0]
    col = edge_index[1]
    pad = EPAD - E
    spread = jax.lax.iota(jnp.int32, pad)
    srcp = jnp.concatenate([row, spread % N])
    dstp = jnp.concatenate([col, N + spread % (NPAD - N)])
    colg = jnp.concatenate([col, spread % N])
    dst32 = dstp.reshape(32, 40, 128)
    idxsd = jnp.stack([srcp.reshape(16, 80, 128),
                       dstp.reshape(16, 80, 128)], axis=2)
    src32 = srcp.reshape(32, 5120)
    col32 = colg.reshape(32, 5120)

    b1r = b1.reshape(1, DH)
    b2r = b2.reshape(1, DH)
    wa = jnp.pad(Wc[:DH], ((0, 0), (0, TPC - Wc.shape[1])))
    wb = jnp.pad(Wc[DH:], ((0, 0), (0, TPC - Wc.shape[1])))
    bca = jnp.pad(bc, (0, TPC - bc.shape[0])).reshape(1, TPC)

    degp = _deg_call(dst32)
    dinv = _dinv_call(degp)
    y1c = _mm1_call(x, W1, dinv)
    agg1 = _agg_call(y1c, idxsd)
    y2c = _mm2_call(dinv, b1r, W2, agg1, y1c)
    agg2 = _agg_call(y2c, idxsd)
    atab, btab = _mm3_call(dinv, b2r, wa, wb, bca, agg2, y2c)
    oute = _cls_call(atab.reshape(N * TPC), btab.reshape(N * TPC), src32, col32)
    return oute.reshape(EPAD, TPC)[:E, :3]


# speedup vs baseline: 13.8063x; 1.0840x over previous
import functools

import jax
import jax.numpy as jnp
from jax import lax
from jax.experimental import pallas as pl
from jax.experimental.pallas import tpu as pltpu
from jax.experimental.pallas import tpu_sc as plsc

N = 10000
NPAD = 10240
E = 160000
EPAD = 163840
DIN = 256
DH = 512
CW = 128
NCHUNK = DH // CW
NC, NS = 2, 16
TPC = 4

_f32 = jnp.float32


def _sc_mesh():
    return plsc.VectorSubcoreMesh(core_axis_name="c", subcore_axis_name="s")


_SC_PARAMS = pltpu.CompilerParams(needs_layout_passes=False)


def _deg_body(dst_hbm, out_hbm, idx_v, ones_v, zb_v, acc_sh):
    c = lax.axis_index("c")
    s = lax.axis_index("s")
    wid = c * NS + s
    for i in range(8):
        ones_v[pl.ds(i * 16, 16)] = jnp.full((16,), 1.0, _f32)
    for i in range(40):
        zb_v[pl.ds(i * 16, 16)] = jnp.zeros((16,), _f32)
    pltpu.sync_copy(dst_hbm.at[wid], idx_v)
    pltpu.sync_copy(zb_v, acc_sh.at[pl.ds(s * 640, 640)])
    plsc.subcore_barrier()

    @pl.loop(0, 40)
    def _(j):
        pltpu.sync_copy(ones_v, acc_sh.at[idx_v.at[j]], add=True)

    plsc.subcore_barrier()
    pltpu.sync_copy(acc_sh.at[pl.ds(s * 640, 640)],
                    out_hbm.at[c, pl.ds(s * 640, 640)])


def _deg_call(dst32):
    k = pl.kernel(
        _deg_body,
        out_type=jax.ShapeDtypeStruct((NC, NPAD), _f32),
        mesh=_sc_mesh(),
        compiler_params=_SC_PARAMS,
        scratch_types=[
            pltpu.VMEM((40, 128), jnp.int32),
            pltpu.VMEM((128,), _f32),
            pltpu.VMEM((640,), _f32),
            pltpu.VMEM_SHARED((NPAD,), _f32),
        ],
    )
    return k(dst32)


_NT = 80


def _agg_body(y0, y1, y2, y3, idxsd, a0, a1, a2, a3,
              islots, gb0, gb1, zb, isems, gsems, ssems, acc_sh):
    gbufs = (gb0, gb1)
    c = lax.axis_index("c")
    s = lax.axis_index("s")

    @pl.loop(0, 32)
    def _(r):
        for kk in range(8):
            zb[r, pl.ds(kk * 16, 16)] = jnp.zeros((16,), _f32)

    def run_chunk(ytab, aout):
        @pl.loop(0, 20)
        def _(i):
            pltpu.sync_copy(zb, acc_sh.at[pl.ds(s * 640 + i * 32, 32)])

        plsc.subcore_barrier()

        def start_idx(t, k):
            pltpu.async_copy(idxsd.at[s, t], islots.at[k], isems[k])

        def wait_idx(t, k):
            pltpu.make_async_copy(idxsd.at[s, t], islots.at[k],
                                  isems[k]).wait()

        def start_gather(t, b, k):
            pltpu.async_copy(ytab.at[islots.at[k, 0]], gbufs[b], gsems[b])

        def wait_gather(b, k):
            pltpu.make_async_copy(ytab.at[islots.at[k, 0]], gbufs[b],
                                  gsems[b]).wait()

        def start_scatter(b, k):
            pltpu.async_copy(gbufs[b], acc_sh.at[islots.at[k, 1]], ssems[b],
                             add=True)

        def wait_scatter(b, k):
            pltpu.make_async_copy(gbufs[b], acc_sh.at[islots.at[k, 1]],
                                  ssems[b]).wait()

        for k in range(4):
            start_idx(k, k)
        wait_idx(0, 0)
        start_gather(0, 0, 0)

        @pl.loop(0, _NT // 4)
        def _(g):
            for u in range(4):
                b = u % 2
                k0 = u
                t = g * 4 + u
                if u == 0:
                    @pl.when(g > 0)
                    def _():
                        wait_scatter(1 - b, (u - 1) % 4)
                else:
                    wait_scatter(1 - b, (u - 1) % 4)
                if u == 3:
                    @pl.when(g < _NT // 4 - 1)
                    def _():
                        wait_idx(t + 1, (u + 1) % 4)
                        start_gather(t + 1, 1 - b, (u + 1) % 4)
                else:
                    wait_idx(t + 1, (u + 1) % 4)
                    start_gather(t + 1, 1 - b, (u + 1) % 4)
                wait_gather(b, k0)
                start_scatter(b, k0)
                if u == 0:
                    @pl.when(g > 0)
                    def _():
                        start_idx(t + 3, (u + 3) % 4)
                else:
                    @pl.when(t + 3 < _NT)
                    def _():
                        start_idx(t + 3, (u + 3) % 4)

        wait_scatter(1, 3)
        plsc.subcore_barrier()
        pltpu.sync_copy(acc_sh.at[pl.ds(s * 640, 640)],
                        aout.at[pl.ds(s * 640, 640)])
        plsc.subcore_barrier()

    @pl.when(c == 0)
    def _():
        run_chunk(y0, a0)
        run_chunk(y1, a1)

    @pl.when(c == 1)
    def _():
        run_chunk(y2, a2)
        run_chunk(y3, a3)


def _agg_call(ycs, idxsd):
    k = pl.kernel(
        _agg_body,
        out_type=[jax.ShapeDtypeStruct((NPAD, CW), _f32)] * 4,
        mesh=_sc_mesh(),
        compiler_params=_SC_PARAMS,
        scratch_types=[
            pltpu.VMEM((4, 2, 128), jnp.int32),
            pltpu.VMEM((128, CW), _f32),
            pltpu.VMEM((128, CW), _f32),
            pltpu.VMEM((32, CW), _f32),
            [pltpu.SemaphoreType.DMA] * 4,
            [pltpu.SemaphoreType.DMA] * 2,
            [pltpu.SemaphoreType.DMA] * 2,
            pltpu.VMEM_SHARED((NPAD, CW), _f32),
        ],
    )
    return k(ycs[0], ycs[1], ycs[2], ycs[3], idxsd)


def _cls_body(atab, btab, rowg, colg, out3, atv, btv, ridx, cidx, obuf):
    c = lax.axis_index("c")
    s = lax.axis_index("s")
    wid = c * NS + s
    pltpu.sync_copy(rowg.at[wid], ridx)
    pltpu.sync_copy(colg.at[wid], cidx)
    pltpu.sync_copy(atab, atv)
    pltpu.sync_copy(btab, btv)

    @pl.loop(0, 40)
    def _(j):
        for k in range(8):
            eA = ridx[pl.ds(j * 128 + k * 16, 16)] * TPC
            eB = cidx[pl.ds(j * 128 + k * 16, 16)] * TPC
            for col in range(3):
                sv = (plsc.load_gather(atv, [eA + col]) +
                      plsc.load_gather(btv, [eB + col]))
                obuf[col, pl.ds(j * 128 + k * 16, 16)] = sv

    for col in range(3):
        pltpu.sync_copy(obuf.at[pl.ds(col, 1)],
                        out3.at[pl.ds(col, 1), pl.ds(wid * 5120, 5120)])


def _cls_call(atab, btab, row32, col32):
    k = pl.kernel(
        _cls_body,
        out_type=jax.ShapeDtypeStruct((3, EPAD), _f32),
        mesh=_sc_mesh(),
        compiler_params=_SC_PARAMS,
        scratch_types=[
            pltpu.VMEM((N * TPC,), _f32),
            pltpu.VMEM((N * TPC,), _f32),
            pltpu.VMEM((5120,), jnp.int32),
            pltpu.VMEM((5120,), jnp.int32),
            pltpu.VMEM((3, 5120), _f32),
        ],
    )
    return k(atab, btab, row32, col32)


def _tr_body(i_ref, o_ref):
    o_ref[...] = i_ref[...].T


def _tr_call(out3):
    return pl.pallas_call(
        _tr_body,
        grid=(125,),
        in_specs=[pl.BlockSpec((3, 1280), lambda i: (0, i))],
        out_specs=pl.BlockSpec((1280, 3), lambda i: (i, 0)),
        out_shape=jax.ShapeDtypeStruct((E, 3), _f32),
    )(out3)


def _dinv_body(degp_ref, out_ref):
    d = 1.0 + degp_ref[0, :] + degp_ref[1, :]
    out_ref[...] = lax.rsqrt(d)[:, None]


def _dinv_call(degp):
    return pl.pallas_call(
        _dinv_body,
        grid=(10,),
        in_specs=[pl.BlockSpec((NC, 1024), lambda i: (0, i))],
        out_specs=pl.BlockSpec((1024, 1), lambda i: (i, 0)),
        out_shape=jax.ShapeDtypeStruct((NPAD, 1), _f32),
    )(degp)


def _mm1_body(x_ref, w_ref, dv_ref, o0, o1, o2, o3):
    y = jnp.dot(x_ref[...], w_ref[...], preferred_element_type=_f32)
    y = y * dv_ref[...]
    o0[...] = y[:, 0:128]
    o1[...] = y[:, 128:256]
    o2[...] = y[:, 256:384]
    o3[...] = y[:, 384:512]


def _mm1_call(x, W1, dinv):
    return pl.pallas_call(
        _mm1_body,
        grid=(25,),
        in_specs=[
            pl.BlockSpec((400, DIN), lambda i: (i, 0)),
            pl.BlockSpec((DIN, DH), lambda i: (0, 0)),
            pl.BlockSpec((400, 1), lambda i: (i, 0)),
        ],
        out_specs=[pl.BlockSpec((400, CW), lambda i: (i, 0))] * 4,
        out_shape=[jax.ShapeDtypeStruct((N, CW), _f32)] * 4,
    )(x, W1, dinv)


def _mm2_body(dv_ref, b_ref, w_ref, a0, a1, a2, a3, y0, y1, y2, y3,
              o0, o1, o2, o3):
    dv = dv_ref[...]
    h = jnp.concatenate(
        [a0[...] + y0[...], a1[...] + y1[...],
         a2[...] + y2[...], a3[...] + y3[...]], axis=1)
    h = jnp.tanh(h * dv + b_ref[...])
    y = jnp.dot(h, w_ref[...], preferred_element_type=_f32) * dv
    o0[...] = y[:, 0:128]
    o1[...] = y[:, 128:256]
    o2[...] = y[:, 256:384]
    o3[...] = y[:, 384:512]


def _mm2_call(dinv, b1r, W2, aggs, ycs):
    return pl.pallas_call(
        _mm2_body,
        grid=(25,),
        in_specs=[
            pl.BlockSpec((400, 1), lambda i: (i, 0)),
            pl.BlockSpec((1, DH), lambda i: (0, 0)),
            pl.BlockSpec((DH, DH), lambda i: (0, 0)),
        ] + [pl.BlockSpec((400, CW), lambda i: (i, 0))] * 8,
        out_specs=[pl.BlockSpec((400, CW), lambda i: (i, 0))] * 4,
        out_shape=[jax.ShapeDtypeStruct((N, CW), _f32)] * 4,
    )(dinv, b1r, W2, *aggs, *ycs)


def _mm3_body(dv_ref, b_ref, wa_ref, wb_ref, bca_ref,
              a0, a1, a2, a3, y0, y1, y2, y3, oa, ob):
    dv = dv_ref[...]
    h = jnp.concatenate(
        [a0[...] + y0[...], a1[...] + y1[...],
         a2[...] + y2[...], a3[...] + y3[...]], axis=1)
    h = jnp.clip(h * dv + b_ref[...], 0.0, 6.0)
    oa[...] = jnp.dot(h, wa_ref[...], preferred_element_type=_f32) + bca_ref[...]
    ob[...] = jnp.dot(h, wb_ref[...], preferred_element_type=_f32)


def _mm3_call(dinv, b2r, wa, wb, bca, aggs, ycs):
    return pl.pallas_call(
        _mm3_body,
        grid=(25,),
        in_specs=[
            pl.BlockSpec((400, 1), lambda i: (i, 0)),
            pl.BlockSpec((1, DH), lambda i: (0, 0)),
            pl.BlockSpec((DH, TPC), lambda i: (0, 0)),
            pl.BlockSpec((DH, TPC), lambda i: (0, 0)),
            pl.BlockSpec((1, TPC), lambda i: (0, 0)),
        ] + [pl.BlockSpec((400, CW), lambda i: (i, 0))] * 8,
        out_specs=[pl.BlockSpec((400, TPC), lambda i: (i, 0))] * 2,
        out_shape=[jax.ShapeDtypeStruct((N, TPC), _f32)] * 2,
    )(dinv, b2r, wa, wb, bca, *aggs, *ycs)


def kernel(x, edge_index, edge_attr, W1, b1, W2, b2, Wc, bc):
    row = edge_index[0]
    col = edge_index[1]
    pad = EPAD - E
    spread = jax.lax.iota(jnp.int32, pad)
    srcp = jnp.concatenate([row, spread % N])
    dstp = jnp.concatenate([col, N + spread % (NPAD - N)])
    colg = jnp.concatenate([col, spread % N])
    dst32 = dstp.reshape(32, 40, 128)
    idxsd = jnp.stack([srcp.reshape(16, 80, 128),
                       dstp.reshape(16, 80, 128)], axis=2)
    src32 = srcp.reshape(32, 5120)
    col32 = colg.reshape(32, 5120)

    b1r = b1.reshape(1, DH)
    b2r = b2.reshape(1, DH)
    wa = jnp.pad(Wc[:DH], ((0, 0), (0, TPC - Wc.shape[1])))
    wb = jnp.pad(Wc[DH:], ((0, 0), (0, TPC - Wc.shape[1])))
    bca = jnp.pad(bc, (0, TPC - bc.shape[0])).reshape(1, TPC)

    degp = _deg_call(dst32)
    dinv = _dinv_call(degp)
    y1c = _mm1_call(x, W1, dinv)
    agg1 = _agg_call(y1c, idxsd)
    y2c = _mm2_call(dinv, b1r, W2, agg1, y1c)
    agg2 = _agg_call(y2c, idxsd)
    atab, btab = _mm3_call(dinv, b2r, wa, wb, bca, agg2, y2c)
    out3 = _cls_call(atab.reshape(N * TPC), btab.reshape(N * TPC), src32, col32)
    return _tr_call(out3)


# speedup vs baseline: 14.0517x; 1.0178x over previous
import functools

import jax
import jax.numpy as jnp
from jax import lax
from jax.experimental import pallas as pl
from jax.experimental.pallas import tpu as pltpu
from jax.experimental.pallas import tpu_sc as plsc

N = 10000
NPAD = 10240
E = 160000
EPAD = 163840
DIN = 256
DH = 512
CW = 128
NCHUNK = DH // CW
NC, NS = 2, 16
TPC = 4

_f32 = jnp.float32


def _sc_mesh():
    return plsc.VectorSubcoreMesh(core_axis_name="c", subcore_axis_name="s")


_SC_PARAMS = pltpu.CompilerParams(needs_layout_passes=False)


def _deg_body(dst_hbm, out_hbm, idx_v, ones_v, zb_v, acc_sh):
    c = lax.axis_index("c")
    s = lax.axis_index("s")
    wid = c * NS + s
    for i in range(8):
        ones_v[pl.ds(i * 16, 16)] = jnp.full((16,), 1.0, _f32)
    for i in range(40):
        zb_v[pl.ds(i * 16, 16)] = jnp.zeros((16,), _f32)
    pltpu.sync_copy(dst_hbm.at[wid], idx_v)
    pltpu.sync_copy(zb_v, acc_sh.at[pl.ds(s * 640, 640)])
    plsc.subcore_barrier()

    @pl.loop(0, 40)
    def _(j):
        pltpu.sync_copy(ones_v, acc_sh.at[idx_v.at[j]], add=True)

    plsc.subcore_barrier()
    pltpu.sync_copy(acc_sh.at[pl.ds(s * 640, 640)],
                    out_hbm.at[c, pl.ds(s * 640, 640)])


def _deg_call(dst32):
    k = pl.kernel(
        _deg_body,
        out_type=jax.ShapeDtypeStruct((NC, NPAD), _f32),
        mesh=_sc_mesh(),
        compiler_params=_SC_PARAMS,
        scratch_types=[
            pltpu.VMEM((40, 128), jnp.int32),
            pltpu.VMEM((128,), _f32),
            pltpu.VMEM((640,), _f32),
            pltpu.VMEM_SHARED((NPAD,), _f32),
        ],
    )
    return k(dst32)


_NT = 80


def _agg_body(y0, y1, y2, y3, idxsd, a0, a1, a2, a3,
              islots, gb0, gb1, zb, isems, gsems, ssems, acc_sh):
    gbufs = (gb0, gb1)
    c = lax.axis_index("c")
    s = lax.axis_index("s")

    @pl.loop(0, 32)
    def _(r):
        for kk in range(8):
            zb[r, pl.ds(kk * 16, 16)] = jnp.zeros((16,), _f32)

    def run_chunk(ytab, aout):
        @pl.loop(0, 20)
        def _(i):
            pltpu.sync_copy(zb, acc_sh.at[pl.ds(s * 640 + i * 32, 32)])

        plsc.subcore_barrier()

        def start_idx(t, k):
            pltpu.async_copy(idxsd.at[s, t], islots.at[k], isems[k])

        def wait_idx(t, k):
            pltpu.make_async_copy(idxsd.at[s, t], islots.at[k],
                                  isems[k]).wait()

        def start_gather(t, b, k):
            pltpu.async_copy(ytab.at[islots.at[k, 0]], gbufs[b], gsems[b])

        def wait_gather(b, k):
            pltpu.make_async_copy(ytab.at[islots.at[k, 0]], gbufs[b],
                                  gsems[b]).wait()

        def start_scatter(b, k):
            pltpu.async_copy(gbufs[b], acc_sh.at[islots.at[k, 1]], ssems[b],
                             add=True)

        def wait_scatter(b, k):
            pltpu.make_async_copy(gbufs[b], acc_sh.at[islots.at[k, 1]],
                                  ssems[b]).wait()

        for k in range(4):
            start_idx(k, k)
        wait_idx(0, 0)
        start_gather(0, 0, 0)

        @pl.loop(0, _NT // 4)
        def _(g):
            for u in range(4):
                b = u % 2
                k0 = u
                t = g * 4 + u
                if u == 0:
                    @pl.when(g > 0)
                    def _():
                        wait_scatter(1 - b, (u - 1) % 4)
                else:
                    wait_scatter(1 - b, (u - 1) % 4)
                if u == 3:
                    @pl.when(g < _NT // 4 - 1)
                    def _():
                        wait_idx(t + 1, (u + 1) % 4)
                        start_gather(t + 1, 1 - b, (u + 1) % 4)
                else:
                    wait_idx(t + 1, (u + 1) % 4)
                    start_gather(t + 1, 1 - b, (u + 1) % 4)
                wait_gather(b, k0)
                start_scatter(b, k0)
                if u == 0:
                    @pl.when(g > 0)
                    def _():
                        start_idx(t + 3, (u + 3) % 4)
                else:
                    @pl.when(t + 3 < _NT)
                    def _():
                        start_idx(t + 3, (u + 3) % 4)

        wait_scatter(1, 3)
        plsc.subcore_barrier()
        pltpu.sync_copy(acc_sh.at[pl.ds(s * 640, 640)],
                        aout.at[pl.ds(s * 640, 640)])
        plsc.subcore_barrier()

    @pl.when(c == 0)
    def _():
        run_chunk(y0, a0)
        run_chunk(y1, a1)

    @pl.when(c == 1)
    def _():
        run_chunk(y2, a2)
        run_chunk(y3, a3)


def _agg_call(ycs, idxsd):
    k = pl.kernel(
        _agg_body,
        out_type=[jax.ShapeDtypeStruct((NPAD, CW), _f32)] * 4,
        mesh=_sc_mesh(),
        compiler_params=_SC_PARAMS,
        scratch_types=[
            pltpu.VMEM((4, 2, 128), jnp.int32),
            pltpu.VMEM((128, CW), _f32),
            pltpu.VMEM((128, CW), _f32),
            pltpu.VMEM((32, CW), _f32),
            [pltpu.SemaphoreType.DMA] * 4,
            [pltpu.SemaphoreType.DMA] * 2,
            [pltpu.SemaphoreType.DMA] * 2,
            pltpu.VMEM_SHARED((NPAD, CW), _f32),
        ],
    )
    return k(ycs[0], ycs[1], ycs[2], ycs[3], idxsd)


def _cls_body(atab, btab, rowg, colg, out3, atv, btv, ridx, cidx, obuf):
    c = lax.axis_index("c")
    s = lax.axis_index("s")
    wid = c * NS + s
    pltpu.sync_copy(rowg.at[wid], ridx)
    pltpu.sync_copy(colg.at[wid], cidx)
    pltpu.sync_copy(atab, atv)
    pltpu.sync_copy(btab, btv)

    @pl.loop(0, 40)
    def _(j):
        for k in range(8):
            eA = ridx[pl.ds(j * 128 + k * 16, 16)] * TPC
            eB = cidx[pl.ds(j * 128 + k * 16, 16)] * TPC
            for col in range(3):
                sv = (plsc.load_gather(atv, [eA + col]) +
                      plsc.load_gather(btv, [eB + col]))
                obuf[col, pl.ds(j * 128 + k * 16, 16)] = sv

    for col in range(3):
        pltpu.sync_copy(obuf.at[pl.ds(col, 1)],
                        out3.at[pl.ds(col, 1), pl.ds(wid * 5120, 5120)])


def _cls_call(atab, btab, row32, col32):
    k = pl.kernel(
        _cls_body,
        out_type=jax.ShapeDtypeStruct((3, EPAD), _f32),
        mesh=_sc_mesh(),
        compiler_params=_SC_PARAMS,
        scratch_types=[
            pltpu.VMEM((N * TPC,), _f32),
            pltpu.VMEM((N * TPC,), _f32),
            pltpu.VMEM((5120,), jnp.int32),
            pltpu.VMEM((5120,), jnp.int32),
            pltpu.VMEM((3, 5120), _f32),
        ],
    )
    return k(atab, btab, row32, col32)


def _tr_body(i_ref, o_ref):
    ident = jnp.eye(3, dtype=_f32)
    o_ref[...] = lax.dot_general(
        i_ref[...], ident, (((0,), (0,)), ((), ())),
        precision=lax.Precision.HIGHEST, preferred_element_type=_f32)


def _tr_call(out3):
    return pl.pallas_call(
        _tr_body,
        grid=(25,),
        in_specs=[pl.BlockSpec((3, 6400), lambda i: (0, i))],
        out_specs=pl.BlockSpec((6400, 3), lambda i: (i, 0)),
        out_shape=jax.ShapeDtypeStruct((E, 3), _f32),
    )(out3)


def _dinv_body(degp_ref, out_ref):
    d = 1.0 + degp_ref[0, :] + degp_ref[1, :]
    out_ref[...] = lax.rsqrt(d)[:, None]


def _dinv_call(degp):
    return pl.pallas_call(
        _dinv_body,
        grid=(10,),
        in_specs=[pl.BlockSpec((NC, 1024), lambda i: (0, i))],
        out_specs=pl.BlockSpec((1024, 1), lambda i: (i, 0)),
        out_shape=jax.ShapeDtypeStruct((NPAD, 1), _f32),
    )(degp)


def _mm1_body(x_ref, w_ref, dv_ref, o0, o1, o2, o3):
    y = jnp.dot(x_ref[...], w_ref[...], preferred_element_type=_f32)
    y = y * dv_ref[...]
    o0[...] = y[:, 0:128]
    o1[...] = y[:, 128:256]
    o2[...] = y[:, 256:384]
    o3[...] = y[:, 384:512]


def _mm1_call(x, W1, dinv):
    return pl.pallas_call(
        _mm1_body,
        grid=(25,),
        in_specs=[
            pl.BlockSpec((400, DIN), lambda i: (i, 0)),
            pl.BlockSpec((DIN, DH), lambda i: (0, 0)),
            pl.BlockSpec((400, 1), lambda i: (i, 0)),
        ],
        out_specs=[pl.BlockSpec((400, CW), lambda i: (i, 0))] * 4,
        out_shape=[jax.ShapeDtypeStruct((N, CW), _f32)] * 4,
    )(x, W1, dinv)


def _mm2_body(dv_ref, b_ref, w_ref, a0, a1, a2, a3, y0, y1, y2, y3,
              o0, o1, o2, o3):
    dv = dv_ref[...]
    h = jnp.concatenate(
        [a0[...] + y0[...], a1[...] + y1[...],
         a2[...] + y2[...], a3[...] + y3[...]], axis=1)
    h = jnp.tanh(h * dv + b_ref[...])
    y = jnp.dot(h, w_ref[...], preferred_element_type=_f32) * dv
    o0[...] = y[:, 0:128]
    o1[...] = y[:, 128:256]
    o2[...] = y[:, 256:384]
    o3[...] = y[:, 384:512]


def _mm2_call(dinv, b1r, W2, aggs, ycs):
    return pl.pallas_call(
        _mm2_body,
        grid=(25,),
        in_specs=[
            pl.BlockSpec((400, 1), lambda i: (i, 0)),
            pl.BlockSpec((1, DH), lambda i: (0, 0)),
            pl.BlockSpec((DH, DH), lambda i: (0, 0)),
        ] + [pl.BlockSpec((400, CW), lambda i: (i, 0))] * 8,
        out_specs=[pl.BlockSpec((400, CW), lambda i: (i, 0))] * 4,
        out_shape=[jax.ShapeDtypeStruct((N, CW), _f32)] * 4,
    )(dinv, b1r, W2, *aggs, *ycs)


def _mm3_body(dv_ref, b_ref, wa_ref, wb_ref, bca_ref,
              a0, a1, a2, a3, y0, y1, y2, y3, oa, ob):
    dv = dv_ref[...]
    h = jnp.concatenate(
        [a0[...] + y0[...], a1[...] + y1[...],
         a2[...] + y2[...], a3[...] + y3[...]], axis=1)
    h = jnp.clip(h * dv + b_ref[...], 0.0, 6.0)
    oa[...] = jnp.dot(h, wa_ref[...], preferred_element_type=_f32) + bca_ref[...]
    ob[...] = jnp.dot(h, wb_ref[...], preferred_element_type=_f32)


def _mm3_call(dinv, b2r, wa, wb, bca, aggs, ycs):
    return pl.pallas_call(
        _mm3_body,
        grid=(25,),
        in_specs=[
            pl.BlockSpec((400, 1), lambda i: (i, 0)),
            pl.BlockSpec((1, DH), lambda i: (0, 0)),
            pl.BlockSpec((DH, TPC), lambda i: (0, 0)),
            pl.BlockSpec((DH, TPC), lambda i: (0, 0)),
            pl.BlockSpec((1, TPC), lambda i: (0, 0)),
        ] + [pl.BlockSpec((400, CW), lambda i: (i, 0))] * 8,
        out_specs=[pl.BlockSpec((400, TPC), lambda i: (i, 0))] * 2,
        out_shape=[jax.ShapeDtypeStruct((N, TPC), _f32)] * 2,
    )(dinv, b2r, wa, wb, bca, *aggs, *ycs)


def kernel(x, edge_index, edge_attr, W1, b1, W2, b2, Wc, bc):
    row = edge_index[0]
    col = edge_index[1]
    pad = EPAD - E
    spread = jax.lax.iota(jnp.int32, pad)
    srcp = jnp.concatenate([row, spread % N])
    dstp = jnp.concatenate([col, N + spread % (NPAD - N)])
    colg = jnp.concatenate([col, spread % N])
    dst32 = dstp.reshape(32, 40, 128)
    idxsd = jnp.stack([srcp.reshape(16, 80, 128),
                       dstp.reshape(16, 80, 128)], axis=2)
    src32 = srcp.reshape(32, 5120)
    col32 = colg.reshape(32, 5120)

    b1r = b1.reshape(1, DH)
    b2r = b2.reshape(1, DH)
    wa = jnp.pad(Wc[:DH], ((0, 0), (0, TPC - Wc.shape[1])))
    wb = jnp.pad(Wc[DH:], ((0, 0), (0, TPC - Wc.shape[1])))
    bca = jnp.pad(bc, (0, TPC - bc.shape[0])).reshape(1, TPC)

    degp = _deg_call(dst32)
    dinv = _dinv_call(degp)
    y1c = _mm1_call(x, W1, dinv)
    agg1 = _agg_call(y1c, idxsd)
    y2c = _mm2_call(dinv, b1r, W2, agg1, y1c)
    agg2 = _agg_call(y2c, idxsd)
    atab, btab = _mm3_call(dinv, b2r, wa, wb, bca, agg2, y2c)
    out3 = _cls_call(atab.reshape(N * TPC), btab.reshape(N * TPC), src32, col32)
    return _tr_call(out3)
